# baseline (speedup 1.0000x reference)
import functools

import jax
import jax.numpy as jnp
from jax import lax
from jax.experimental import pallas as pl
from jax.experimental.pallas import tpu as pltpu
from jax.experimental.pallas import tpu_sc as plsc

N_NODES = 10000
N_EDGES = 320000
D = 128

NC = 2
NS = 16
NW = NC * NS
L = 16

CHUNK = 128
CHUNKS_PER_TILE = 80
GROUP = 8
GROUPS = CHUNKS_PER_TILE // GROUP
G0 = 18
G1 = 2 * GROUPS - G0
GMAX = max(G0, G1)
H0 = 16
H1 = 2 * GROUPS - H0
HMAX = max(H0, H1)
E_PAD = NW * CHUNKS_PER_TILE * CHUNK
EROWS = E_PAD // CHUNK
ROWS_PAD = 10112
STRIPE = ROWS_PAD // NS

NPG = 10240
NODES_PER_TILE = NPG // NW
GCHUNK = 40
GCHUNKS = NODES_PER_TILE // GCHUNK

_MESH = plsc.VectorSubcoreMesh(core_axis_name="c", subcore_axis_name="s")


@functools.partial(
    pl.kernel,
    mesh=_MESH,
    out_type=[
        jax.ShapeDtypeStruct((ROWS_PAD, D), jnp.float32),
        jax.ShapeDtypeStruct((ROWS_PAD, D), jnp.float32),
        jax.ShapeDtypeStruct((ROWS_PAD, D), jnp.float32),
        jax.ShapeDtypeStruct((ROWS_PAD, D), jnp.float32),
    ],
    scratch_types=[
        pltpu.VMEM((2, GROUP, CHUNK), jnp.int32),
        pltpu.VMEM((CHUNK, D), jnp.float32),
        pltpu.VMEM((CHUNK, D), jnp.float32),
        pltpu.VMEM_SHARED((ROWS_PAD, D), jnp.float32),
        pltpu.SemaphoreType.DMA,
        pltpu.SemaphoreType.DMA,
    ],
)
def _sc_scatter(edges3d, feat, zsum, ones_hbm,
                sums0, sums1, deg0, deg1,
                idx_v, rows_a, rows_b, acc, sem_a, sem_b):
    c = lax.axis_index("c")
    s = lax.axis_index("s")
    w = s * NC + c

    stripe = pl.ds(s * STRIPE, STRIPE)

    pltpu.sync_copy(zsum.at[stripe], acc.at[stripe])

    plsc.subcore_barrier()

    base_group = jnp.where(c == 0, s * G0, NS * G0 + s * G1)
    ngroups = jnp.where(c == 0, G0, G1)

    def group_body(g, carry):
        @pl.when(g < ngroups)
        def _():
            slab = pl.ds((base_group + g) * GROUP, GROUP)
            pltpu.sync_copy(edges3d.at[:, slab], idx_v)
            pltpu.async_copy(feat.at[idx_v.at[0, 0]], rows_a, sem_a)

            def pair_body(j2, carry2):
                c0 = 2 * j2
                pltpu.make_async_copy(feat.at[idx_v.at[0, c0]], rows_a, sem_a).wait()
                pltpu.async_copy(feat.at[idx_v.at[0, c0 + 1]], rows_b, sem_b)
                pltpu.sync_copy(rows_a, acc.at[idx_v.at[1, c0]], add=True)
                pltpu.make_async_copy(feat.at[idx_v.at[0, c0 + 1]], rows_b,
                                      sem_b).wait()

                @pl.when(j2 < GROUP // 2 - 1)
                def _():
                    pltpu.async_copy(feat.at[idx_v.at[0, c0 + 2]], rows_a, sem_a)

                pltpu.sync_copy(rows_b, acc.at[idx_v.at[1, c0 + 1]], add=True)
                return carry2

            lax.fori_loop(0, GROUP // 2, pair_body, 0)

        return carry

    lax.fori_loop(0, GMAX, group_body, 0)

    plsc.subcore_barrier()

    @pl.when(c == 0)
    def _():
        pltpu.sync_copy(acc.at[stripe], sums0.at[stripe])

    @pl.when(c == 1)
    def _():
        pltpu.sync_copy(acc.at[stripe], sums1.at[stripe])

    pltpu.sync_copy(zsum.at[stripe], acc.at[stripe])
    pltpu.sync_copy(ones_hbm, rows_a)

    plsc.subcore_barrier()

    base2 = jnp.where(c == 0, s * H0, NS * H0 + s * H1)
    ngroups2 = jnp.where(c == 0, H0, H1)

    def group2_body(g, carry):
        @pl.when(g < ngroups2)
        def _():
            slab = pl.ds((base2 + g) * GROUP, GROUP)
            pltpu.sync_copy(edges3d.at[:, slab], idx_v)

            def pair2_body(j2, carry2):
                c0 = 2 * j2
                ca = pltpu.async_copy(rows_a, acc.at[idx_v.at[1, c0]], sem_a,
                                      add=True)
                cb = pltpu.async_copy(rows_a, acc.at[idx_v.at[1, c0 + 1]], sem_b,
                                      add=True)
                ca.wait()
                cb.wait()
                return carry2

            lax.fori_loop(0, GROUP // 2, pair2_body, 0)

        return carry

    lax.fori_loop(0, HMAX, group2_body, 0)

    plsc.subcore_barrier()

    @pl.when(c == 0)
    def _():
        pltpu.sync_copy(acc.at[stripe], deg0.at[stripe])

    @pl.when(c == 1)
    def _():
        pltpu.sync_copy(acc.at[stripe], deg1.at[stripe])


@functools.partial(
    pl.kernel,
    mesh=_MESH,
    out_type=[
        jax.ShapeDtypeStruct((NPG, D), jnp.float32),
        jax.ShapeDtypeStruct((NPG, D), jnp.float32),
    ],
    scratch_types=[
        pltpu.VMEM((GCHUNK,), jnp.int32),
        pltpu.VMEM((GCHUNK, D), jnp.float32),
        pltpu.VMEM((GCHUNK, D), jnp.float32),
        pltpu.SemaphoreType.DMA,
        pltpu.SemaphoreType.DMA,
    ],
)
def _sc_gather(nodes1d, feat, mean, self_o, neigh_o, idx_v, fa, fb, sem_a, sem_b):
    c = lax.axis_index("c")
    s = lax.axis_index("s")
    w = s * NC + c

    def body(j, carry):
        nbase = w * NODES_PER_TILE + j * GCHUNK
        out_row = pl.ds(nbase, GCHUNK)
        pltpu.sync_copy(nodes1d.at[pl.ds(nbase, GCHUNK)], idx_v)
        pltpu.async_copy(feat.at[idx_v], fa, sem_a)
        pltpu.async_copy(mean.at[idx_v], fb, sem_b)
        pltpu.make_async_copy(feat.at[idx_v], fa, sem_a).wait()
        pltpu.make_async_copy(mean.at[idx_v], fb, sem_b).wait()
        pltpu.sync_copy(fa, self_o.at[out_row])
        pltpu.sync_copy(fb, neigh_o.at[out_row])
        return carry

    lax.fori_loop(0, GCHUNKS, body, 0)


def _tc_mean(s0_ref, s1_ref, d0_ref, d1_ref, mean_ref):
    deg = d0_ref[:, 0:1] + d1_ref[:, 0:1]
    inv = 1.0 / jnp.maximum(deg, 1.0)
    mean_ref[...] = (s0_ref[...] + s1_ref[...]) * inv


def _tc_mlp(self_ref, neigh_ref, w1_ref, b1_ref, out_ref):
    acc = jnp.dot(self_ref[...], w1_ref[0:D, :], preferred_element_type=jnp.float32)
    acc = acc + jnp.dot(neigh_ref[...], w1_ref[D:2 * D, :],
                        preferred_element_type=jnp.float32)
    out_ref[...] = jnp.maximum(acc + b1_ref[...], 0.0)


def kernel(nodes, edge_index, feat_table, W1, b1):
    src = edge_index[0]
    dst = edge_index[1]
    pad_dst = N_NODES + jnp.arange(E_PAD - N_EDGES, dtype=jnp.int32) % (
        ROWS_PAD - N_NODES)
    src_p = jnp.concatenate([src, jnp.zeros((E_PAD - N_EDGES,), jnp.int32)])
    dst_p = jnp.concatenate([dst, pad_dst])
    edges3d = jnp.stack([src_p, dst_p]).reshape(2, EROWS, CHUNK)
    nodes_p = jnp.concatenate([nodes, jnp.zeros((NPG - N_NODES,), jnp.int32)])

    zsum = jnp.zeros((ROWS_PAD, D), jnp.float32)
    ones = jnp.ones((CHUNK, D), jnp.float32)

    sums0, sums1, deg0, deg1 = _sc_scatter(edges3d, feat_table, zsum, ones)

    MBM = 1264
    mean = pl.pallas_call(
        _tc_mean,
        grid=(ROWS_PAD // MBM,),
        in_specs=[
            pl.BlockSpec((MBM, D), lambda i: (i, 0)),
            pl.BlockSpec((MBM, D), lambda i: (i, 0)),
            pl.BlockSpec((MBM, D), lambda i: (i, 0)),
            pl.BlockSpec((MBM, D), lambda i: (i, 0)),
        ],
        out_specs=pl.BlockSpec((MBM, D), lambda i: (i, 0)),
        out_shape=jax.ShapeDtypeStruct((ROWS_PAD, D), jnp.float32),
    )(sums0, sums1, deg0, deg1)

    self_f, neigh = _sc_gather(nodes_p, feat_table, mean)

    b1r = b1.reshape(1, D)
    MB = 1024
    out = pl.pallas_call(
        _tc_mlp,
        grid=(NPG // MB,),
        in_specs=[
            pl.BlockSpec((MB, D), lambda i: (i, 0)),
            pl.BlockSpec((MB, D), lambda i: (i, 0)),
            pl.BlockSpec((2 * D, D), lambda i: (0, 0)),
            pl.BlockSpec((1, D), lambda i: (0, 0)),
        ],
        out_specs=pl.BlockSpec((MB, D), lambda i: (i, 0)),
        out_shape=jax.ShapeDtypeStruct((NPG, D), jnp.float32),
    )(self_f, neigh, W1, b1r)

    return out[:N_NODES]


# speedup vs baseline: 1.0145x; 1.0145x over previous
import functools

import jax
import jax.numpy as jnp
from jax import lax
from jax.experimental import pallas as pl
from jax.experimental.pallas import tpu as pltpu
from jax.experimental.pallas import tpu_sc as plsc

N_NODES = 10000
N_EDGES = 320000
D = 128

NC = 2
NS = 16
NW = NC * NS
L = 16

CHUNK = 128
CHUNKS_PER_TILE = 80
GROUP = 8
GROUPS = CHUNKS_PER_TILE // GROUP
G0 = 18
G1 = 2 * GROUPS - G0
GMAX = max(G0, G1)
H0 = 14
H1 = 2 * GROUPS - H0
HMAX = max(H0, H1)
E_PAD = NW * CHUNKS_PER_TILE * CHUNK
EROWS = E_PAD // CHUNK
ROWS_PAD = 10112
STRIPE = ROWS_PAD // NS

NPG = 10240
NODES_PER_TILE = NPG // NW
GCHUNK = 40
GCHUNKS = NODES_PER_TILE // GCHUNK

_MESH = plsc.VectorSubcoreMesh(core_axis_name="c", subcore_axis_name="s")


@functools.partial(
    pl.kernel,
    mesh=_MESH,
    out_type=[
        jax.ShapeDtypeStruct((ROWS_PAD, D), jnp.float32),
        jax.ShapeDtypeStruct((ROWS_PAD, D), jnp.float32),
        jax.ShapeDtypeStruct((ROWS_PAD, D), jnp.float32),
        jax.ShapeDtypeStruct((ROWS_PAD, D), jnp.float32),
    ],
    scratch_types=[
        pltpu.VMEM((2, GROUP, CHUNK), jnp.int32),
        pltpu.VMEM((CHUNK, D), jnp.float32),
        pltpu.VMEM((CHUNK, D), jnp.float32),
        pltpu.VMEM_SHARED((ROWS_PAD, D), jnp.float32),
        pltpu.SemaphoreType.DMA,
        pltpu.SemaphoreType.DMA,
    ],
)
def _sc_scatter(edges3d, feat, zsum, ones_hbm,
                sums0, sums1, deg0, deg1,
                idx_v, rows_a, rows_b, acc, sem_a, sem_b):
    c = lax.axis_index("c")
    s = lax.axis_index("s")
    w = s * NC + c

    stripe = pl.ds(s * STRIPE, STRIPE)

    pltpu.sync_copy(zsum.at[stripe], acc.at[stripe])

    plsc.subcore_barrier()

    base_group = jnp.where(c == 0, s * G0, NS * G0 + s * G1)
    ngroups = jnp.where(c == 0, G0, G1)

    def group_body(g, carry):
        @pl.when(g < ngroups)
        def _():
            slab = pl.ds((base_group + g) * GROUP, GROUP)
            pltpu.sync_copy(edges3d.at[:, slab], idx_v)
            pltpu.async_copy(feat.at[idx_v.at[0, 0]], rows_a, sem_a)

            def pair_body(j2, carry2):
                c0 = 2 * j2
                pltpu.make_async_copy(feat.at[idx_v.at[0, c0]], rows_a, sem_a).wait()
                pltpu.async_copy(feat.at[idx_v.at[0, c0 + 1]], rows_b, sem_b)
                pltpu.sync_copy(rows_a, acc.at[idx_v.at[1, c0]], add=True)
                pltpu.make_async_copy(feat.at[idx_v.at[0, c0 + 1]], rows_b,
                                      sem_b).wait()

                @pl.when(j2 < GROUP // 2 - 1)
                def _():
                    pltpu.async_copy(feat.at[idx_v.at[0, c0 + 2]], rows_a, sem_a)

                pltpu.sync_copy(rows_b, acc.at[idx_v.at[1, c0 + 1]], add=True)
                return carry2

            lax.fori_loop(0, GROUP // 2, pair_body, 0)

        return carry

    lax.fori_loop(0, GMAX, group_body, 0)

    plsc.subcore_barrier()

    @pl.when(c == 0)
    def _():
        pltpu.sync_copy(acc.at[stripe], sums0.at[stripe])

    @pl.when(c == 1)
    def _():
        pltpu.sync_copy(acc.at[stripe], sums1.at[stripe])

    pltpu.sync_copy(zsum.at[stripe], acc.at[stripe])
    pltpu.sync_copy(ones_hbm, rows_a)

    plsc.subcore_barrier()

    base2 = jnp.where(c == 0, s * H0, NS * H0 + s * H1)
    ngroups2 = jnp.where(c == 0, H0, H1)

    def group2_body(g, carry):
        @pl.when(g < ngroups2)
        def _():
            slab = pl.ds((base2 + g) * GROUP, GROUP)
            pltpu.sync_copy(edges3d.at[:, slab], idx_v)

            def pair2_body(j2, carry2):
                c0 = 2 * j2
                ca = pltpu.async_copy(rows_a, acc.at[idx_v.at[1, c0]], sem_a,
                                      add=True)
                cb = pltpu.async_copy(rows_a, acc.at[idx_v.at[1, c0 + 1]], sem_b,
                                      add=True)
                ca.wait()
                cb.wait()
                return carry2

            lax.fori_loop(0, GROUP // 2, pair2_body, 0)

        return carry

    lax.fori_loop(0, HMAX, group2_body, 0)

    plsc.subcore_barrier()

    @pl.when(c == 0)
    def _():
        pltpu.sync_copy(acc.at[stripe], deg0.at[stripe])

    @pl.when(c == 1)
    def _():
        pltpu.sync_copy(acc.at[stripe], deg1.at[stripe])


@functools.partial(
    pl.kernel,
    mesh=_MESH,
    out_type=[
        jax.ShapeDtypeStruct((NPG, D), jnp.float32),
        jax.ShapeDtypeStruct((NPG, D), jnp.float32),
    ],
    scratch_types=[
        pltpu.VMEM((GCHUNK,), jnp.int32),
        pltpu.VMEM((GCHUNK, D), jnp.float32),
        pltpu.VMEM((GCHUNK, D), jnp.float32),
        pltpu.SemaphoreType.DMA,
        pltpu.SemaphoreType.DMA,
    ],
)
def _sc_gather(nodes1d, feat, mean, self_o, neigh_o, idx_v, fa, fb, sem_a, sem_b):
    c = lax.axis_index("c")
    s = lax.axis_index("s")
    w = s * NC + c

    def body(j, carry):
        nbase = w * NODES_PER_TILE + j * GCHUNK
        out_row = pl.ds(nbase, GCHUNK)
        pltpu.sync_copy(nodes1d.at[pl.ds(nbase, GCHUNK)], idx_v)
        pltpu.async_copy(feat.at[idx_v], fa, sem_a)
        pltpu.async_copy(mean.at[idx_v], fb, sem_b)
        pltpu.make_async_copy(feat.at[idx_v], fa, sem_a).wait()
        pltpu.make_async_copy(mean.at[idx_v], fb, sem_b).wait()
        pltpu.sync_copy(fa, self_o.at[out_row])
        pltpu.sync_copy(fb, neigh_o.at[out_row])
        return carry

    lax.fori_loop(0, GCHUNKS, body, 0)


def _tc_mean(s0_ref, s1_ref, d0_ref, d1_ref, mean_ref):
    deg = d0_ref[:, 0:1] + d1_ref[:, 0:1]
    inv = 1.0 / jnp.maximum(deg, 1.0)
    mean_ref[...] = (s0_ref[...] + s1_ref[...]) * inv


def _tc_mlp(self_ref, neigh_ref, w1_ref, b1_ref, out_ref):
    acc = jnp.dot(self_ref[...], w1_ref[0:D, :], preferred_element_type=jnp.float32)
    acc = acc + jnp.dot(neigh_ref[...], w1_ref[D:2 * D, :],
                        preferred_element_type=jnp.float32)
    out_ref[...] = jnp.maximum(acc + b1_ref[...], 0.0)


def kernel(nodes, edge_index, feat_table, W1, b1):
    src = edge_index[0]
    dst = edge_index[1]
    pad_dst = N_NODES + jnp.arange(E_PAD - N_EDGES, dtype=jnp.int32) % (
        ROWS_PAD - N_NODES)
    src_p = jnp.concatenate([src, jnp.zeros((E_PAD - N_EDGES,), jnp.int32)])
    dst_p = jnp.concatenate([dst, pad_dst])
    edges3d = jnp.stack([src_p, dst_p]).reshape(2, EROWS, CHUNK)
    nodes_p = jnp.concatenate([nodes, jnp.zeros((NPG - N_NODES,), jnp.int32)])

    zsum = jnp.zeros((ROWS_PAD, D), jnp.float32)
    ones = jnp.ones((CHUNK, D), jnp.float32)

    sums0, sums1, deg0, deg1 = _sc_scatter(edges3d, feat_table, zsum, ones)

    MBM = 1264
    mean = pl.pallas_call(
        _tc_mean,
        grid=(ROWS_PAD // MBM,),
        in_specs=[
            pl.BlockSpec((MBM, D), lambda i: (i, 0)),
            pl.BlockSpec((MBM, D), lambda i: (i, 0)),
            pl.BlockSpec((MBM, D), lambda i: (i, 0)),
            pl.BlockSpec((MBM, D), lambda i: (i, 0)),
        ],
        out_specs=pl.BlockSpec((MBM, D), lambda i: (i, 0)),
        out_shape=jax.ShapeDtypeStruct((ROWS_PAD, D), jnp.float32),
    )(sums0, sums1, deg0, deg1)

    self_f, neigh = _sc_gather(nodes_p, feat_table, mean)

    b1r = b1.reshape(1, D)
    MB = 1024
    out = pl.pallas_call(
        _tc_mlp,
        grid=(NPG // MB,),
        in_specs=[
            pl.BlockSpec((MB, D), lambda i: (i, 0)),
            pl.BlockSpec((MB, D), lambda i: (i, 0)),
            pl.BlockSpec((2 * D, D), lambda i: (0, 0)),
            pl.BlockSpec((1, D), lambda i: (0, 0)),
        ],
        out_specs=pl.BlockSpec((MB, D), lambda i: (i, 0)),
        out_shape=jax.ShapeDtypeStruct((NPG, D), jnp.float32),
    )(self_f, neigh, W1, b1r)

    return out[:N_NODES]
